# baseline (speedup 1.0000x reference)
import functools

import jax
import jax.numpy as jnp
from jax.experimental import pallas as pl
from jax.experimental.pallas import tpu as pltpu

N_OUT = 255
TILE_M = 16384
TILE_N = 128


def _cde_kernel(x_ref, w_ref, b_ref, o_ref):
    x16 = x_ref[...].astype(jnp.bfloat16)
    acc = jnp.dot(x16, w_ref[...], preferred_element_type=jnp.float32)
    o_ref[...] = (acc + b_ref[...])[:, : o_ref.shape[-1]].astype(o_ref.dtype)


@jax.jit
def _forward(x, w16, b_pad):
    B, d_in = x.shape
    tm = min(TILE_M, B)
    grid = (pl.cdiv(B, tm), pl.cdiv(N_OUT, TILE_N))
    return pl.pallas_call(
        _cde_kernel,
        out_shape=jax.ShapeDtypeStruct((B, N_OUT), x.dtype),
        grid=grid,
        in_specs=[
            pl.BlockSpec((tm, d_in), lambda i, j: (i, 0)),
            pl.BlockSpec((d_in, TILE_N), lambda i, j: (0, j)),
            pl.BlockSpec((1, TILE_N), lambda i, j: (0, j)),
        ],
        out_specs=pl.BlockSpec((tm, TILE_N), lambda i, j: (i, j)),
        compiler_params=pltpu.CompilerParams(
            dimension_semantics=("parallel", "arbitrary"),
        ),
    )(x, w16, b_pad)


def kernel(x, w_t_pad, b_pad):
    return _forward(x, w_t_pad.astype(jnp.bfloat16), b_pad)


# speedup vs baseline: 1.3449x; 1.3449x over previous
import functools

import jax
import jax.numpy as jnp
from jax.experimental import pallas as pl
from jax.experimental.pallas import tpu as pltpu

N_OUT = 255
TILE_M = 8192


def _cde_kernel(x_ref, w_ref, b_ref, o_ref):
    acc = jnp.dot(x_ref[...], w_ref[...], preferred_element_type=jnp.float32)
    o_ref[...] = (acc + b_ref[...])[:, : o_ref.shape[-1]].astype(o_ref.dtype)


@jax.jit
def _forward(x, w_t_pad, b_pad):
    B, d_in = x.shape
    n_pad = w_t_pad.shape[1]
    tm = min(TILE_M, B)
    grid = (pl.cdiv(B, tm),)
    return pl.pallas_call(
        _cde_kernel,
        out_shape=jax.ShapeDtypeStruct((B, N_OUT), x.dtype),
        grid=grid,
        in_specs=[
            pl.BlockSpec((tm, d_in), lambda i: (i, 0)),
            pl.BlockSpec((d_in, n_pad), lambda i: (0, 0)),
            pl.BlockSpec((1, n_pad), lambda i: (0, 0)),
        ],
        out_specs=pl.BlockSpec((tm, N_OUT), lambda i: (i, 0)),
        compiler_params=pltpu.CompilerParams(
            dimension_semantics=("parallel",),
        ),
    )(x, w_t_pad, b_pad)


def kernel(x, w_t_pad, b_pad):
    return _forward(x, w_t_pad, b_pad)


# speedup vs baseline: 1.3817x; 1.0274x over previous
import functools

import jax
import jax.numpy as jnp
from jax.experimental import pallas as pl
from jax.experimental.pallas import tpu as pltpu

N_OUT = 255
TILE_M = 12288


def _cde_kernel(x_ref, w_ref, b_ref, o_ref):
    acc = jnp.dot(x_ref[...], w_ref[...], preferred_element_type=jnp.float32)
    o_ref[...] = (acc + b_ref[...])[:, : o_ref.shape[-1]].astype(o_ref.dtype)


@jax.jit
def _forward(x, w_t_pad, b_pad):
    B, d_in = x.shape
    n_pad = w_t_pad.shape[1]
    tm = min(TILE_M, B)
    grid = (pl.cdiv(B, tm),)
    return pl.pallas_call(
        _cde_kernel,
        out_shape=jax.ShapeDtypeStruct((B, N_OUT), x.dtype),
        grid=grid,
        in_specs=[
            pl.BlockSpec((tm, d_in), lambda i: (i, 0)),
            pl.BlockSpec((d_in, n_pad), lambda i: (0, 0)),
            pl.BlockSpec((1, n_pad), lambda i: (0, 0)),
        ],
        out_specs=pl.BlockSpec((tm, N_OUT), lambda i: (i, 0)),
        compiler_params=pltpu.CompilerParams(
            dimension_semantics=("parallel",),
        ),
    )(x, w_t_pad, b_pad)


def kernel(x, w_t_pad, b_pad):
    return _forward(x, w_t_pad, b_pad)
